# baseline (speedup 1.0000x reference)
import jax
import jax.numpy as jnp
from jax.experimental import pallas as pl
from jax.experimental.pallas import tpu as pltpu
from jax.experimental.pallas import tpu_sc as plsc

B, C, HW = 16, 256, 1024
M = 1024
D = 256
T = B * HW

_NC, _NS = 2, 16
_NW = _NC * _NS
_BPW = T // _NW
_CHUNK = 256


def _vq_body(x_ref, e_ref, idx_ref, loss_ref, perp_ref, cnt_acc, min_acc):
    b = pl.program_id(0)
    xb = x_ref[...]
    emb = e_ref[...]
    cross_m2 = jnp.dot(-2.0 * emb, xb,
                       preferred_element_type=jnp.float32)
    e_sq = jnp.sum(emb * emb, axis=1, keepdims=True)
    x_sq = jnp.sum(xb * xb, axis=0, keepdims=True)
    dist = (e_sq + x_sq) + cross_m2
    minv = jnp.min(dist, axis=0, keepdims=True)
    miota = jax.lax.broadcasted_iota(jnp.int32, (M, HW), 0)
    idx = jnp.min(jnp.where(dist == minv, miota, jnp.int32(2**30)),
                  axis=0, keepdims=True)
    idx_ref[...] = idx
    one_hot_t = jnp.where(miota == idx, 1.0, 0.0)
    cnt = jnp.sum(one_hot_t, axis=1, keepdims=True)
    msum = jnp.sum(minv).reshape(1, 1)

    @pl.when(b == 0)
    def _():
        cnt_acc[...] = cnt
        min_acc[...] = msum

    @pl.when(b != 0)
    def _():
        cnt_acc[...] += cnt
        min_acc[...] += msum

    @pl.when(b == B - 1)
    def _():
        loss_ref[...] = (0.25 / (T * D)) * min_acc[...]
        p = cnt_acc[...] * (1.0 / T)
        ent = jnp.sum(p * jnp.log(p + 1e-10))
        perp_ref[...] = jnp.exp(-ent).reshape(1, 1)


def _sc_gather_body(emb_hbm, idx_hbm, out_hbm, idx_v, rows_v, sem):
    wid = jax.lax.axis_index("s") * _NC + jax.lax.axis_index("c")
    base = wid * _BPW
    pltpu.sync_copy(idx_hbm.at[pl.ds(base, _BPW)], idx_v)
    for chunk in range(_BPW // _CHUNK):
        pltpu.async_copy(
            emb_hbm.at[idx_v.at[pl.ds(chunk * _CHUNK, _CHUNK)]],
            rows_v, sem).wait()
        pltpu.sync_copy(rows_v,
                        out_hbm.at[pl.ds(base + chunk * _CHUNK, _CHUNK)])


def kernel(x, embedding):
    x3 = x.reshape(B, C, HW)
    emb = embedding.reshape(M, D)
    idx3, loss2, perp2 = pl.pallas_call(
        _vq_body,
        grid=(B,),
        in_specs=[
            pl.BlockSpec((None, C, HW), lambda b: (b, 0, 0)),
            pl.BlockSpec((M, D), lambda b: (0, 0)),
        ],
        out_specs=[
            pl.BlockSpec((None, 1, HW), lambda b: (b, 0, 0)),
            pl.BlockSpec((1, 1), lambda b: (0, 0)),
            pl.BlockSpec((1, 1), lambda b: (0, 0)),
        ],
        out_shape=[
            jax.ShapeDtypeStruct((B, 1, HW), jnp.int32),
            jax.ShapeDtypeStruct((1, 1), jnp.float32),
            jax.ShapeDtypeStruct((1, 1), jnp.float32),
        ],
        scratch_shapes=[
            pltpu.VMEM((M, 1), jnp.float32),
            pltpu.VMEM((1, 1), jnp.float32),
        ],
    )(x3, emb)
    indices = idx3.reshape(1, T)
    idx_flat = idx3.reshape(T)

    sc_gather = pl.kernel(
        _sc_gather_body,
        out_type=jax.ShapeDtypeStruct((T, D), jnp.float32),
        mesh=plsc.VectorSubcoreMesh(core_axis_name="c", subcore_axis_name="s"),
        scratch_types=[
            pltpu.VMEM((_BPW,), jnp.int32),
            pltpu.VMEM((_CHUNK, D), jnp.float32),
            pltpu.SemaphoreType.DMA,
        ],
    )
    quant = sc_gather(emb, idx_flat)
    out = quant.reshape(B, HW, D).transpose(0, 2, 1).reshape(B, C, 32, 32)
    return (out, loss2.reshape(()), perp2.reshape(()), indices)
